# baseline (speedup 1.0000x reference)
import jax
import jax.numpy as jnp
from jax import lax
from jax.experimental import pallas as pl
from jax.experimental.pallas import tpu as pltpu
from jax.experimental.pallas import tpu_sc as plsc

ROWS = 64
COLS = 8192
K = 3
LANES = 16
NUM_CORES = 2
NUM_SUBCORES = 16
NUM_WORKERS = NUM_CORES * NUM_SUBCORES
ROWS_PER_WORKER = ROWS // NUM_WORKERS
CHUNKS = COLS // LANES
STREAMS = 4
STEPS = CHUNKS // STREAMS


def _insert(acc, cvals, cidx):
    v1, i1, v2, i2, v3, i3 = acc
    gt1 = cvals > v1
    t = jnp.minimum(cvals, v1)
    it = jnp.where(gt1, i1, cidx)
    v1 = jnp.maximum(cvals, v1)
    i1 = jnp.where(gt1, cidx, i1)
    gt2 = t > v2
    t2 = jnp.minimum(t, v2)
    it2 = jnp.where(gt2, i2, it)
    v2 = jnp.maximum(t, v2)
    i2 = jnp.where(gt2, it, i2)
    gt3 = t2 > v3
    v3 = jnp.maximum(t2, v3)
    i3 = jnp.where(gt3, it2, i3)
    return v1, i1, v2, i2, v3, i3


def _insert_tie(acc, cvals, cidx):
    v1, i1, v2, i2, v3, i3 = acc
    gt1 = (cvals > v1) | ((cvals == v1) & (cidx < i1))
    t = jnp.where(gt1, v1, cvals)
    it = jnp.where(gt1, i1, cidx)
    nv1 = jnp.where(gt1, cvals, v1)
    i1 = jnp.where(gt1, cidx, i1)
    gt2 = (t > v2) | ((t == v2) & (it < i2))
    t2 = jnp.where(gt2, v2, t)
    it2 = jnp.where(gt2, i2, it)
    v2 = jnp.where(gt2, t, v2)
    i2 = jnp.where(gt2, it, i2)
    gt3 = (t2 > v3) | ((t2 == v3) & (it2 < i3))
    v3 = jnp.where(gt3, t2, v3)
    i3 = jnp.where(gt3, it2, i3)
    return nv1, i1, v2, i2, v3, i3


def _merge(a, b):
    for lv in range(3):
        a = _insert_tie(a, b[2 * lv], b[2 * lv + 1])
    return a


def _body(x_hbm, out_hbm, rows_v, resv_v, resi_v, sem):
    c = lax.axis_index("c")
    s = lax.axis_index("s")
    wid = s * NUM_CORES + c

    lane = lax.broadcasted_iota(jnp.int32, (LANES,), 0)
    neg = jnp.full((LANES,), -jnp.inf, jnp.float32)
    zero_i = jnp.zeros((LANES,), jnp.int32)
    big = jnp.full((LANES,), jnp.int32(2**30), jnp.int32)

    base = wid * ROWS_PER_WORKER
    cps = [
        pltpu.make_async_copy(
            x_hbm.at[base + r], rows_v.at[pl.ds(r * COLS, COLS)], sem)
        for r in range(ROWS_PER_WORKER)
    ]
    for cp in cps:
        cp.start()
    for cp in cps:
        cp.wait()

    def row_body(r, _):
        roff = r * COLS
        init = tuple((neg, zero_i, neg, zero_i, neg, zero_i)[i % 6]
                     for i in range(6 * STREAMS))

        def step(j, carry):
            accs = [carry[6 * q:6 * q + 6] for q in range(STREAMS)]
            out = []
            coff = j * (STREAMS * LANES)
            for q in range(STREAMS):
                cvals = rows_v[pl.ds(roff + coff + q * LANES, LANES)]
                cidx = lane + (coff + q * LANES)
                out.extend(_insert(accs[q], cvals, cidx))
            return tuple(out)

        flat = lax.fori_loop(0, STEPS, step, init)
        accs = [flat[6 * q:6 * q + 6] for q in range(STREAMS)]
        while len(accs) > 1:
            accs = [_merge(accs[i], accs[i + 1])
                    for i in range(0, len(accs), 2)]
        v1, i1, v2, i2, v3, i3 = accs[0]

        out_v, out_i = [], []
        for _k in range(K):
            m = jnp.max(v1)
            sel = jnp.min(jnp.where(v1 == m, i1, big))
            out_v.append(m)
            out_i.append(sel)
            hit = (v1 == m) & (i1 == sel)
            v1 = jnp.where(hit, v2, v1)
            i1 = jnp.where(hit, i2, i1)
            v2 = jnp.where(hit, v3, v2)
            i2 = jnp.where(hit, i3, i2)
            v3 = jnp.where(hit, neg, v3)

        resv = jnp.where(lane == 0, out_v[0],
                         jnp.where(lane == 1, out_v[1],
                                   jnp.where(lane == 2, out_v[2], 0.0)))
        resi = jnp.where(lane == 0, out_i[0],
                         jnp.where(lane == 1, out_i[1],
                                   jnp.where(lane == 2, out_i[2], 0)))
        resv_v[...] = plsc.bitcast(resv.astype(jnp.float32), jnp.int32)
        resi_v[...] = resi.astype(jnp.int32)
        pltpu.sync_copy(resv_v, out_hbm.at[base + r])
        pltpu.sync_copy(resi_v, out_hbm.at[ROWS + base + r])
        return 0

    lax.fori_loop(0, ROWS_PER_WORKER, row_body, 0)


@jax.jit
def _topk_sc(x):
    mesh = plsc.VectorSubcoreMesh(core_axis_name="c", subcore_axis_name="s")
    fn = pl.kernel(
        _body,
        out_type=jax.ShapeDtypeStruct((2 * ROWS, LANES), jnp.int32),
        mesh=mesh,
        scratch_types=[
            pltpu.VMEM((ROWS_PER_WORKER * COLS,), jnp.float32),
            pltpu.VMEM((LANES,), jnp.int32),
            pltpu.VMEM((LANES,), jnp.int32),
            pltpu.SemaphoreType.DMA,
        ],
        compiler_params=pltpu.CompilerParams(needs_layout_passes=False),
    )
    return fn(x)


def kernel(x):
    out = _topk_sc(x)
    vals = lax.bitcast_convert_type(out[:ROWS, :K], jnp.float32)
    idx = out[ROWS:, :K]
    return vals, idx


# speedup vs baseline: 1.0369x; 1.0369x over previous
import jax
import jax.numpy as jnp
from jax import lax
from jax.experimental import pallas as pl
from jax.experimental.pallas import tpu as pltpu
from jax.experimental.pallas import tpu_sc as plsc

ROWS = 64
COLS = 8192
K = 3
LANES = 16
NUM_CORES = 2
NUM_SUBCORES = 16
NUM_WORKERS = NUM_CORES * NUM_SUBCORES
ROWS_PER_WORKER = ROWS // NUM_WORKERS
CHUNKS = COLS // LANES
PAIRS = CHUNKS // 2
STREAMS = 4
PSTEPS = PAIRS // STREAMS


def _insert(acc, cvals, cidx):
    v1, i1, v2, i2, v3, i3 = acc
    gt1 = cvals > v1
    t = jnp.minimum(cvals, v1)
    it = jnp.where(gt1, i1, cidx)
    v1 = jnp.maximum(cvals, v1)
    i1 = jnp.where(gt1, cidx, i1)
    gt2 = t > v2
    t2 = jnp.minimum(t, v2)
    it2 = jnp.where(gt2, i2, it)
    v2 = jnp.maximum(t, v2)
    i2 = jnp.where(gt2, it, i2)
    gt3 = t2 > v3
    v3 = jnp.maximum(t2, v3)
    i3 = jnp.where(gt3, it2, i3)
    return v1, i1, v2, i2, v3, i3


def _insert_tie(acc, cvals, cidx):
    v1, i1, v2, i2, v3, i3 = acc
    gt1 = (cvals > v1) | ((cvals == v1) & (cidx < i1))
    t = jnp.where(gt1, v1, cvals)
    it = jnp.where(gt1, i1, cidx)
    nv1 = jnp.where(gt1, cvals, v1)
    i1 = jnp.where(gt1, cidx, i1)
    gt2 = (t > v2) | ((t == v2) & (it < i2))
    t2 = jnp.where(gt2, v2, t)
    it2 = jnp.where(gt2, i2, it)
    v2 = jnp.where(gt2, t, v2)
    i2 = jnp.where(gt2, it, i2)
    gt3 = (t2 > v3) | ((t2 == v3) & (it2 < i3))
    v3 = jnp.where(gt3, t2, v3)
    i3 = jnp.where(gt3, it2, i3)
    return nv1, i1, v2, i2, v3, i3


def _merge(a, b):
    for lv in range(3):
        a = _insert_tie(a, b[2 * lv], b[2 * lv + 1])
    return a


def _body(x_hbm, out_hbm, rows_v, resv_v, resi_v, sem):
    c = lax.axis_index("c")
    s = lax.axis_index("s")
    wid = s * NUM_CORES + c

    lane = lax.broadcasted_iota(jnp.int32, (LANES,), 0)
    neg = jnp.full((LANES,), -jnp.inf, jnp.float32)
    zero_i = jnp.zeros((LANES,), jnp.int32)
    big = jnp.full((LANES,), jnp.int32(2**30), jnp.int32)

    base = wid * ROWS_PER_WORKER
    cps = [
        pltpu.make_async_copy(
            x_hbm.at[base + r], rows_v.at[pl.ds(r * COLS, COLS)], sem)
        for r in range(ROWS_PER_WORKER)
    ]
    for cp in cps:
        cp.start()
    for cp in cps:
        cp.wait()

    lane16 = lane + 16

    def row_body(r, _):
        roff = r * COLS
        init = tuple((neg, zero_i, neg, zero_i, neg, zero_i)[i % 6]
                     for i in range(6 * STREAMS))

        def step(j, carry):
            accs = [carry[6 * q:6 * q + 6] for q in range(STREAMS)]
            out = []
            for q in range(STREAMS):
                off = (j * STREAMS + q) * (2 * LANES)
                ca = rows_v[pl.ds(roff + off, LANES)]
                cb = rows_v[pl.ds(roff + off + LANES, LANES)]
                ge = ca >= cb
                m = jnp.maximum(ca, cb)
                col = jnp.where(ge, lane, lane16) + off
                out.extend(_insert(accs[q], m, col))
            return tuple(out)

        flat = lax.fori_loop(0, PSTEPS, step, init)
        accs = [flat[6 * q:6 * q + 6] for q in range(STREAMS)]
        while len(accs) > 1:
            accs = [_merge(accs[i], accs[i + 1])
                    for i in range(0, len(accs), 2)]
        v1, i1, v2, i2, v3, i3 = accs[0]

        out_v, out_i = [], []
        for _k in range(K):
            m = jnp.max(v1)
            sel = jnp.min(jnp.where(v1 == m, i1, big))
            out_v.append(m)
            out_i.append(sel)
            hit = (v1 == m) & (i1 == sel)
            v1 = jnp.where(hit, v2, v1)
            i1 = jnp.where(hit, i2, i1)
            v2 = jnp.where(hit, v3, v2)
            i2 = jnp.where(hit, i3, i2)
            v3 = jnp.where(hit, neg, v3)

        pc = [out_i[k] ^ 16 for k in range(K)]
        idxv = jnp.where(lane == 0, pc[0],
                         jnp.where(lane == 1, pc[1],
                                   jnp.where(lane == 2, pc[2], 0))) + roff
        g = plsc.load_gather(rows_v, [idxv])
        zf = jnp.zeros((LANES,), jnp.float32)
        zi = jnp.zeros((LANES,), jnp.int32)
        acc6 = (zf + out_v[0], zi + out_i[0],
                zf + out_v[1], zi + out_i[1],
                zf + out_v[2], zi + out_i[2])
        for k in range(K):
            lval = jnp.max(jnp.where(lane == k, g, neg))
            acc6 = _insert_tie(acc6, zf + lval, zi + pc[k])
        w1, j1, w2, j2, w3, j3 = acc6

        resv = jnp.where(lane == 0, w1,
                         jnp.where(lane == 1, w2,
                                   jnp.where(lane == 2, w3, 0.0)))
        resi = jnp.where(lane == 0, j1,
                         jnp.where(lane == 1, j2,
                                   jnp.where(lane == 2, j3, 0)))
        resv_v[...] = plsc.bitcast(resv.astype(jnp.float32), jnp.int32)
        resi_v[...] = resi.astype(jnp.int32)
        pltpu.sync_copy(resv_v, out_hbm.at[base + r])
        pltpu.sync_copy(resi_v, out_hbm.at[ROWS + base + r])
        return 0

    lax.fori_loop(0, ROWS_PER_WORKER, row_body, 0)


@jax.jit
def _topk_sc(x):
    mesh = plsc.VectorSubcoreMesh(core_axis_name="c", subcore_axis_name="s")
    fn = pl.kernel(
        _body,
        out_type=jax.ShapeDtypeStruct((2 * ROWS, LANES), jnp.int32),
        mesh=mesh,
        scratch_types=[
            pltpu.VMEM((ROWS_PER_WORKER * COLS,), jnp.float32),
            pltpu.VMEM((LANES,), jnp.int32),
            pltpu.VMEM((LANES,), jnp.int32),
            pltpu.SemaphoreType.DMA,
        ],
        compiler_params=pltpu.CompilerParams(needs_layout_passes=False),
    )
    return fn(x)


def kernel(x):
    out = _topk_sc(x)
    vals = lax.bitcast_convert_type(out[:ROWS, :K], jnp.float32)
    idx = out[ROWS:, :K]
    return vals, idx


# speedup vs baseline: 1.0421x; 1.0050x over previous
import jax
import jax.numpy as jnp
from jax import lax
from jax.experimental import pallas as pl
from jax.experimental.pallas import tpu as pltpu
from jax.experimental.pallas import tpu_sc as plsc

ROWS = 64
COLS = 8192
K = 3
LANES = 16
NUM_CORES = 2
NUM_SUBCORES = 16
NUM_WORKERS = NUM_CORES * NUM_SUBCORES
ROWS_PER_WORKER = ROWS // NUM_WORKERS
CHUNKS = COLS // LANES
GROUP = 4
GROUPS = CHUNKS // GROUP
STREAMS = 4
PSTEPS = GROUPS // STREAMS


def _insert(acc, cvals, cidx):
    v1, i1, v2, i2, v3, i3 = acc
    gt1 = cvals > v1
    t = jnp.minimum(cvals, v1)
    it = jnp.where(gt1, i1, cidx)
    v1 = jnp.maximum(cvals, v1)
    i1 = jnp.where(gt1, cidx, i1)
    gt2 = t > v2
    t2 = jnp.minimum(t, v2)
    it2 = jnp.where(gt2, i2, it)
    v2 = jnp.maximum(t, v2)
    i2 = jnp.where(gt2, it, i2)
    gt3 = t2 > v3
    v3 = jnp.maximum(t2, v3)
    i3 = jnp.where(gt3, it2, i3)
    return v1, i1, v2, i2, v3, i3


def _insert_tie(acc, cvals, cidx):
    v1, i1, v2, i2, v3, i3 = acc
    gt1 = (cvals > v1) | ((cvals == v1) & (cidx < i1))
    t = jnp.where(gt1, v1, cvals)
    it = jnp.where(gt1, i1, cidx)
    nv1 = jnp.where(gt1, cvals, v1)
    i1 = jnp.where(gt1, cidx, i1)
    gt2 = (t > v2) | ((t == v2) & (it < i2))
    t2 = jnp.where(gt2, v2, t)
    it2 = jnp.where(gt2, i2, it)
    v2 = jnp.where(gt2, t, v2)
    i2 = jnp.where(gt2, it, i2)
    gt3 = (t2 > v3) | ((t2 == v3) & (it2 < i3))
    v3 = jnp.where(gt3, t2, v3)
    i3 = jnp.where(gt3, it2, i3)
    return nv1, i1, v2, i2, v3, i3


def _merge(a, b):
    for lv in range(3):
        a = _insert_tie(a, b[2 * lv], b[2 * lv + 1])
    return a


def _body(x_hbm, out_hbm, rows_v, resv_v, resi_v, sem):
    c = lax.axis_index("c")
    s = lax.axis_index("s")
    wid = s * NUM_CORES + c

    lane = lax.broadcasted_iota(jnp.int32, (LANES,), 0)
    neg = jnp.full((LANES,), -jnp.inf, jnp.float32)
    zero_i = jnp.zeros((LANES,), jnp.int32)
    big = jnp.full((LANES,), jnp.int32(2**30), jnp.int32)

    base = wid * ROWS_PER_WORKER
    cps = [
        pltpu.make_async_copy(
            x_hbm.at[base + r], rows_v.at[pl.ds(r * COLS, COLS)], sem)
        for r in range(ROWS_PER_WORKER)
    ]
    for cp in cps:
        cp.start()
    for cp in cps:
        cp.wait()

    lane16 = lane + 16
    lane32 = lane + 32
    lane48 = lane + 48

    def row_body(r, _):
        roff = r * COLS
        init = tuple((neg, zero_i, neg, zero_i, neg, zero_i)[i % 6]
                     for i in range(6 * STREAMS))

        def step(j, carry):
            accs = [carry[6 * q:6 * q + 6] for q in range(STREAMS)]
            out = []
            for q in range(STREAMS):
                off = (j * STREAMS + q) * (GROUP * LANES)
                ca = rows_v[pl.ds(roff + off, LANES)]
                cb = rows_v[pl.ds(roff + off + LANES, LANES)]
                cc = rows_v[pl.ds(roff + off + 2 * LANES, LANES)]
                cd = rows_v[pl.ds(roff + off + 3 * LANES, LANES)]
                ge1 = ca >= cb
                m1 = jnp.maximum(ca, cb)
                col1 = jnp.where(ge1, lane, lane16)
                ge2 = cc >= cd
                m2 = jnp.maximum(cc, cd)
                col2 = jnp.where(ge2, lane32, lane48)
                ge = m1 >= m2
                m = jnp.maximum(m1, m2)
                col = jnp.where(ge, col1, col2) + off
                out.extend(_insert(accs[q], m, col))
            return tuple(out)

        flat = lax.fori_loop(0, PSTEPS, step, init)
        accs = [flat[6 * q:6 * q + 6] for q in range(STREAMS)]
        while len(accs) > 1:
            accs = [_merge(accs[i], accs[i + 1])
                    for i in range(0, len(accs), 2)]
        v1, i1, v2, i2, v3, i3 = accs[0]

        out_v, out_i = [], []
        for _k in range(K):
            m = jnp.max(v1)
            sel = jnp.min(jnp.where(v1 == m, i1, big))
            out_v.append(m)
            out_i.append(sel)
            hit = (v1 == m) & (i1 == sel)
            v1 = jnp.where(hit, v2, v1)
            i1 = jnp.where(hit, i2, i1)
            v2 = jnp.where(hit, v3, v2)
            i2 = jnp.where(hit, i3, i2)
            v3 = jnp.where(hit, neg, v3)

        pc = [out_i[k] ^ mask for k in range(K) for mask in (16, 32, 48)]
        idxv = zero_i
        for n in reversed(range(len(pc))):
            idxv = jnp.where(lane == n, pc[n], idxv)
        g = plsc.load_gather(rows_v, [idxv + roff])
        zf = jnp.zeros((LANES,), jnp.float32)
        zi = jnp.zeros((LANES,), jnp.int32)
        acc6 = (zf + out_v[0], zi + out_i[0],
                zf + out_v[1], zi + out_i[1],
                zf + out_v[2], zi + out_i[2])
        for n in range(len(pc)):
            lval = jnp.max(jnp.where(lane == n, g, neg))
            acc6 = _insert_tie(acc6, zf + lval, zi + pc[n])
        w1, j1, w2, j2, w3, j3 = acc6

        resv = jnp.where(lane == 0, w1,
                         jnp.where(lane == 1, w2,
                                   jnp.where(lane == 2, w3, 0.0)))
        resi = jnp.where(lane == 0, j1,
                         jnp.where(lane == 1, j2,
                                   jnp.where(lane == 2, j3, 0)))
        resv_v[...] = plsc.bitcast(resv.astype(jnp.float32), jnp.int32)
        resi_v[...] = resi.astype(jnp.int32)
        pltpu.sync_copy(resv_v, out_hbm.at[base + r])
        pltpu.sync_copy(resi_v, out_hbm.at[ROWS + base + r])
        return 0

    lax.fori_loop(0, ROWS_PER_WORKER, row_body, 0)


@jax.jit
def _topk_sc(x):
    mesh = plsc.VectorSubcoreMesh(core_axis_name="c", subcore_axis_name="s")
    fn = pl.kernel(
        _body,
        out_type=jax.ShapeDtypeStruct((2 * ROWS, LANES), jnp.int32),
        mesh=mesh,
        scratch_types=[
            pltpu.VMEM((ROWS_PER_WORKER * COLS,), jnp.float32),
            pltpu.VMEM((LANES,), jnp.int32),
            pltpu.VMEM((LANES,), jnp.int32),
            pltpu.SemaphoreType.DMA,
        ],
        compiler_params=pltpu.CompilerParams(needs_layout_passes=False),
    )
    return fn(x)


def kernel(x):
    out = _topk_sc(x)
    vals = lax.bitcast_convert_type(out[:ROWS, :K], jnp.float32)
    idx = out[ROWS:, :K]
    return vals, idx
